# baseline (speedup 1.0000x reference)
import jax
import jax.numpy as jnp
from jax import lax
from jax.experimental import pallas as pl
from jax.experimental.pallas import tpu as pltpu
from jax.experimental.pallas import tpu_sc as plsc

N_FACTORS = 32
BATCH = 16384
NC, NS, L = 2, 16, 16
NW = NC * NS
BPW = BATCH // NW
HALF = BPW // 2
HGROUPS = HALF // L
Y_SCALE = 5.0


def _mf_body(users_hbm, items_hbm, uemb_hbm, vemb_hbm, ubias_hbm, ibias_hbm,
             out_hbm, uidx_v, iidx_v, urid_v, irid_v, ubrid_v, ibrid_v,
             urows_v, vrows_v, ub_v, ib_v, out_v, sem_e, sem_b):
    wid = lax.axis_index("s") * NC + lax.axis_index("c")
    base = wid * BPW

    pltpu.sync_copy(users_hbm.at[pl.ds(base, BPW)], uidx_v)
    pltpu.sync_copy(items_hbm.at[pl.ds(base, BPW)], iidx_v)

    def shift(i, _):
        us = uidx_v[pl.ds(i * L, L)]
        vs = iidx_v[pl.ds(i * L, L)]
        urid_v[pl.ds(i * L, L)] = lax.shift_right_logical(us, 2)
        irid_v[pl.ds(i * L, L)] = lax.shift_right_logical(vs, 2)
        ubrid_v[pl.ds(i * L, L)] = lax.shift_right_logical(us, 4)
        ibrid_v[pl.ds(i * L, L)] = lax.shift_right_logical(vs, 4)
        return 0
    lax.fori_loop(0, BPW // L, shift, 0, unroll=False)

    cb1 = pltpu.async_copy(ubias_hbm.at[ubrid_v], ub_v, sem_b)
    cb2 = pltpu.async_copy(ibias_hbm.at[ibrid_v], ib_v, sem_b)
    ce1 = pltpu.async_copy(uemb_hbm.at[urid_v.at[pl.ds(0, HALF)]], urows_v, sem_e)
    ce2 = pltpu.async_copy(vemb_hbm.at[irid_v.at[pl.ds(0, HALF)]], vrows_v, sem_e)

    lane = lax.iota(jnp.int32, L)

    def compute_half(h):
        def group(g, _):
            rows = lane + g * L
            boff = h * HALF + g * L
            ui = uidx_v[pl.ds(boff, L)]
            vi = iidx_v[pl.ds(boff, L)]
            acc = (plsc.load_gather(ub_v, [lane + boff, ui & 15])
                   + plsc.load_gather(ib_v, [lane + boff, vi & 15]))
            ucol = (ui & 3) * 32
            vcol = (vi & 3) * 32
            for f in range(N_FACTORS):
                u = plsc.load_gather(urows_v, [rows, ucol + f])
                v = plsc.load_gather(vrows_v, [rows, vcol + f])
                acc = acc + u * v
            out_v[pl.ds(boff, L)] = Y_SCALE / (1.0 + jnp.exp(-acc))
            return 0
        lax.fori_loop(0, HGROUPS, group, 0, unroll=False)

    ce1.wait()
    ce2.wait()
    cb1.wait()
    cb2.wait()
    compute_half(0)
    ce3 = pltpu.async_copy(uemb_hbm.at[urid_v.at[pl.ds(HALF, HALF)]], urows_v, sem_e)
    ce4 = pltpu.async_copy(vemb_hbm.at[irid_v.at[pl.ds(HALF, HALF)]], vrows_v, sem_e)
    ce3.wait()
    ce4.wait()
    compute_half(1)

    pltpu.sync_copy(out_v, out_hbm.at[pl.ds(base, BPW)])


@jax.jit
def _mf_call(users, items, uemb128, vemb128, ubias16, ibias16):
    kern = pl.kernel(
        _mf_body,
        out_type=jax.ShapeDtypeStruct((BATCH,), jnp.float32),
        mesh=plsc.VectorSubcoreMesh(core_axis_name="c", subcore_axis_name="s"),
        scratch_types=[
            pltpu.VMEM((BPW,), jnp.int32),
            pltpu.VMEM((BPW,), jnp.int32),
            pltpu.VMEM((BPW,), jnp.int32),
            pltpu.VMEM((BPW,), jnp.int32),
            pltpu.VMEM((BPW,), jnp.int32),
            pltpu.VMEM((BPW,), jnp.int32),
            pltpu.VMEM((HALF, 128), jnp.float32),
            pltpu.VMEM((HALF, 128), jnp.float32),
            pltpu.VMEM((BPW, 16), jnp.float32),
            pltpu.VMEM((BPW, 16), jnp.float32),
            pltpu.VMEM((BPW,), jnp.float32),
            pltpu.SemaphoreType.DMA,
            pltpu.SemaphoreType.DMA,
        ],
        compiler_params=pltpu.CompilerParams(
            use_tc_tiling_on_sc=False, needs_layout_passes=False),
    )
    return kern(users, items, uemb128, vemb128, ubias16, ibias16)


def kernel(users, items, user_embedding, item_embedding, user_bias, item_bias):
    users = users.astype(jnp.int32)
    items = items.astype(jnp.int32)
    nu, nf = user_embedding.shape
    ue128 = user_embedding.reshape(nu * nf // 128, 128)
    ie128 = item_embedding.reshape(nu * nf // 128, 128)
    nb = user_bias.shape[0]
    ub16 = user_bias.reshape(nb // 16, 16)
    ib16 = item_bias.reshape(nb // 16, 16)
    return _mf_call(users, items, ue128, ie128, ub16, ib16)


# speedup vs baseline: 1.0071x; 1.0071x over previous
import jax
import jax.numpy as jnp
from jax import lax
from jax.experimental import pallas as pl
from jax.experimental.pallas import tpu as pltpu
from jax.experimental.pallas import tpu_sc as plsc

N_FACTORS = 32
BATCH = 16384
NC, NS, L = 2, 16, 16
NW = NC * NS
BPW = BATCH // NW
GROUPS = BPW // L
Y_SCALE = 5.0


def _mf_body(users_hbm, items_hbm, uemb_hbm, vemb_hbm, ubias_hbm, ibias_hbm,
             out_hbm, uidx_v, iidx_v, urid_v, irid_v, urows_v, vrows_v,
             ub_v, ib_v, out_v, sem):
    wid = lax.axis_index("s") * NC + lax.axis_index("c")
    base = wid * BPW

    pltpu.sync_copy(users_hbm.at[pl.ds(base, BPW)], uidx_v)
    pltpu.sync_copy(items_hbm.at[pl.ds(base, BPW)], iidx_v)

    def shift(i, _):
        urid_v[pl.ds(i * L, L)] = lax.shift_right_logical(uidx_v[pl.ds(i * L, L)], 4)
        irid_v[pl.ds(i * L, L)] = lax.shift_right_logical(iidx_v[pl.ds(i * L, L)], 4)
        return 0
    lax.fori_loop(0, GROUPS, shift, 0, unroll=False)

    c1 = pltpu.async_copy(uemb_hbm.at[uidx_v], urows_v, sem)
    c2 = pltpu.async_copy(vemb_hbm.at[iidx_v], vrows_v, sem)
    c3 = pltpu.async_copy(ubias_hbm.at[urid_v], ub_v, sem)
    c4 = pltpu.async_copy(ibias_hbm.at[irid_v], ib_v, sem)
    c1.wait()
    c2.wait()
    c3.wait()
    c4.wait()

    lane = lax.iota(jnp.int32, L)

    def group(g, _):
        rows = lane + g * L
        acc = (plsc.load_gather(ub_v, [rows, uidx_v[pl.ds(g * L, L)] & 15])
               + plsc.load_gather(ib_v, [rows, iidx_v[pl.ds(g * L, L)] & 15]))
        for f in range(N_FACTORS):
            cols = jnp.full((L,), f, jnp.int32)
            u = plsc.load_gather(urows_v, [rows, cols])
            v = plsc.load_gather(vrows_v, [rows, cols])
            acc = acc + u * v
        out_v[pl.ds(g * L, L)] = Y_SCALE / (1.0 + jnp.exp(-acc))
        return 0

    lax.fori_loop(0, GROUPS, group, 0, unroll=False)
    pltpu.sync_copy(out_v, out_hbm.at[pl.ds(base, BPW)])


@jax.jit
def _mf_call(users, items, uemb, vemb, ubias16, ibias16):
    kern = pl.kernel(
        _mf_body,
        out_type=jax.ShapeDtypeStruct((BATCH,), jnp.float32),
        mesh=plsc.VectorSubcoreMesh(core_axis_name="c", subcore_axis_name="s"),
        scratch_types=[
            pltpu.VMEM((BPW,), jnp.int32),
            pltpu.VMEM((BPW,), jnp.int32),
            pltpu.VMEM((BPW,), jnp.int32),
            pltpu.VMEM((BPW,), jnp.int32),
            pltpu.VMEM((BPW, N_FACTORS), jnp.float32),
            pltpu.VMEM((BPW, N_FACTORS), jnp.float32),
            pltpu.VMEM((BPW, 16), jnp.float32),
            pltpu.VMEM((BPW, 16), jnp.float32),
            pltpu.VMEM((BPW,), jnp.float32),
            pltpu.SemaphoreType.DMA,
        ],
        compiler_params=pltpu.CompilerParams(
            use_tc_tiling_on_sc=False, needs_layout_passes=False),
    )
    return kern(users, items, uemb, vemb, ubias16, ibias16)


def kernel(users, items, user_embedding, item_embedding, user_bias, item_bias):
    users = users.astype(jnp.int32)
    items = items.astype(jnp.int32)
    n = user_bias.shape[0]
    ub16 = user_bias.reshape(n // 16, 16)
    ib16 = item_bias.reshape(n // 16, 16)
    return _mf_call(users, items, user_embedding, item_embedding, ub16, ib16)
